# speedup vs baseline: 1.1961x; 1.1961x over previous
import functools

import jax
import jax.numpy as jnp
from jax import lax
from jax.experimental import pallas as pl
from jax.experimental.pallas import tpu as pltpu

T = 4096
H = 1024
I = 512
E = 8
TOP_K = 2

BT = 256


def _dense_body(x_ref, gw_ref, bias_ref, wgu_ref, wd_ref, sgu_ref, sd_ref, out_ref):
    e = pl.program_id(1)
    x = x_ref[...]

    logits = lax.dot_general(x, gw_ref[...], (((1,), (1,)), ((), ())),
                             preferred_element_type=jnp.float32)
    s = jax.nn.sigmoid(logits)
    sc = s + bias_ref[...]
    ii = lax.broadcasted_iota(jnp.int32, (BT, E), 1)
    m1 = jnp.max(sc, axis=1, keepdims=True)
    i1 = jnp.min(jnp.where(sc >= m1, ii, E), axis=1, keepdims=True)
    sc2 = jnp.where(ii == i1, -jnp.inf, sc)
    m2 = jnp.max(sc2, axis=1, keepdims=True)
    i2 = jnp.min(jnp.where(sc2 >= m2, ii, E), axis=1, keepdims=True)
    s1 = jnp.sum(jnp.where(ii == i1, s, 0.0), axis=1, keepdims=True)
    s2 = jnp.sum(jnp.where(ii == i2, s, 0.0), axis=1, keepdims=True)
    denom = s1 + s2
    w1 = s1 / denom
    w2 = s2 / denom
    w_e = jnp.where(i1 == e, w1, 0.0) + jnp.where(i2 == e, w2, 0.0)

    gu = jnp.dot(x, wgu_ref[0], preferred_element_type=jnp.float32)
    g = gu[:, :I]
    u = gu[:, I:]
    h = jax.nn.silu(g) * u
    oe = jnp.dot(h, wd_ref[0], preferred_element_type=jnp.float32)
    contrib = w_e * oe

    @pl.when(e == 0)
    def _():
        sgu = jnp.dot(x, sgu_ref[...], preferred_element_type=jnp.float32)
        sg = sgu[:, :I]
        su = sgu[:, I:]
        sh = jax.nn.silu(sg) * su
        so = jnp.dot(sh, sd_ref[...], preferred_element_type=jnp.float32)
        out_ref[...] = so + contrib

    @pl.when(e > 0)
    def _():
        out_ref[...] += contrib


@functools.partial(jax.jit, static_argnames=("interpret",))
def _moe_dense(x, gate_weight, bias, w_gate_up, w_down, sgu, sd, interpret=False):
    grid = (T // BT, E)
    return pl.pallas_call(
        _dense_body,
        grid=grid,
        in_specs=[
            pl.BlockSpec((BT, H), lambda t, e: (t, 0)),
            pl.BlockSpec((E, H), lambda t, e: (0, 0)),
            pl.BlockSpec((1, E), lambda t, e: (0, 0)),
            pl.BlockSpec((1, H, 2 * I), lambda t, e: (e, 0, 0)),
            pl.BlockSpec((1, I, H), lambda t, e: (e, 0, 0)),
            pl.BlockSpec((H, 2 * I), lambda t, e: (0, 0)),
            pl.BlockSpec((I, H), lambda t, e: (0, 0)),
        ],
        out_specs=pl.BlockSpec((BT, H), lambda t, e: (t, 0)),
        out_shape=jax.ShapeDtypeStruct((T, H), jnp.float32),
        compiler_params=pltpu.CompilerParams(
            dimension_semantics=("parallel", "arbitrary"),
        ),
        interpret=interpret,
    )(x, gate_weight, bias, w_gate_up, w_down, sgu, sd)


def kernel(hidden_states, gate_weight, correction_bias, w_gate_up, w_down,
           shared_gate_up, shared_down):
    return _moe_dense(hidden_states, gate_weight, correction_bias,
                      w_gate_up, w_down, shared_gate_up, shared_down)


# speedup vs baseline: 1.1994x; 1.0028x over previous
import functools

import jax
import jax.numpy as jnp
from jax import lax
from jax.experimental import pallas as pl
from jax.experimental.pallas import tpu as pltpu

T = 4096
H = 1024
I = 512
E = 8
TOP_K = 2

BT = 256


def _dense_body(x_ref, gw_ref, bias_ref, wgu_ref, wd_ref, sgu_ref, sd_ref, out_ref):
    e = pl.program_id(1)
    x = x_ref[...]

    logits = lax.dot_general(x, gw_ref[...], (((1,), (1,)), ((), ())),
                             preferred_element_type=jnp.float32)
    s = jax.nn.sigmoid(logits)
    sc = s + bias_ref[...]
    ii = lax.broadcasted_iota(jnp.int32, (BT, E), 1)
    m1 = jnp.max(sc, axis=1, keepdims=True)
    i1 = jnp.min(jnp.where(sc >= m1, ii, E), axis=1, keepdims=True)
    sc2 = jnp.where(ii == i1, -jnp.inf, sc)
    m2 = jnp.max(sc2, axis=1, keepdims=True)
    i2 = jnp.min(jnp.where(sc2 >= m2, ii, E), axis=1, keepdims=True)
    s1 = jnp.sum(jnp.where(ii == i1, s, 0.0), axis=1, keepdims=True)
    s2 = jnp.sum(jnp.where(ii == i2, s, 0.0), axis=1, keepdims=True)
    denom = s1 + s2
    w1 = s1 / denom
    w2 = s2 / denom
    w_e = jnp.where(i1 == e, w1, 0.0) + jnp.where(i2 == e, w2, 0.0)

    xb = x.astype(jnp.bfloat16)
    gu = jnp.dot(xb, wgu_ref[0].astype(jnp.bfloat16),
                 preferred_element_type=jnp.float32)
    g = gu[:, :I]
    u = gu[:, I:]
    h = (jax.nn.silu(g) * u).astype(jnp.bfloat16)
    oe = jnp.dot(h, wd_ref[0].astype(jnp.bfloat16),
                 preferred_element_type=jnp.float32)
    contrib = w_e * oe

    @pl.when(e == 0)
    def _():
        sgu = jnp.dot(xb, sgu_ref[...].astype(jnp.bfloat16),
                      preferred_element_type=jnp.float32)
        sg = sgu[:, :I]
        su = sgu[:, I:]
        sh = (jax.nn.silu(sg) * su).astype(jnp.bfloat16)
        so = jnp.dot(sh, sd_ref[...].astype(jnp.bfloat16),
                     preferred_element_type=jnp.float32)
        out_ref[...] = so + contrib

    @pl.when(e > 0)
    def _():
        out_ref[...] += contrib


@functools.partial(jax.jit, static_argnames=("interpret",))
def _moe_dense(x, gate_weight, bias, w_gate_up, w_down, sgu, sd, interpret=False):
    grid = (T // BT, E)
    return pl.pallas_call(
        _dense_body,
        grid=grid,
        in_specs=[
            pl.BlockSpec((BT, H), lambda t, e: (t, 0)),
            pl.BlockSpec((E, H), lambda t, e: (0, 0)),
            pl.BlockSpec((1, E), lambda t, e: (0, 0)),
            pl.BlockSpec((1, H, 2 * I), lambda t, e: (e, 0, 0)),
            pl.BlockSpec((1, I, H), lambda t, e: (e, 0, 0)),
            pl.BlockSpec((H, 2 * I), lambda t, e: (0, 0)),
            pl.BlockSpec((I, H), lambda t, e: (0, 0)),
        ],
        out_specs=pl.BlockSpec((BT, H), lambda t, e: (t, 0)),
        out_shape=jax.ShapeDtypeStruct((T, H), jnp.float32),
        compiler_params=pltpu.CompilerParams(
            dimension_semantics=("parallel", "arbitrary"),
        ),
        interpret=interpret,
    )(x, gate_weight, bias, w_gate_up, w_down, sgu, sd)


def kernel(hidden_states, gate_weight, correction_bias, w_gate_up, w_down,
           shared_gate_up, shared_down):
    return _moe_dense(hidden_states, gate_weight, correction_bias,
                      w_gate_up, w_down, shared_gate_up, shared_down)


# speedup vs baseline: 2.1394x; 1.7837x over previous
import functools

import jax
import jax.numpy as jnp
from jax import lax
from jax.experimental import pallas as pl
from jax.experimental.pallas import tpu as pltpu

T = 4096
H = 1024
I = 512
E = 8
TOP_K = 2

BT = 256


def _dense_body(x_ref, gw_ref, bias_ref, wgu_ref, wd_ref, sgu_ref, sd_ref, out_ref):
    x = x_ref[...]

    logits = lax.dot_general(x, gw_ref[...], (((1,), (1,)), ((), ())),
                             preferred_element_type=jnp.float32)
    s = jax.nn.sigmoid(logits)
    sc = s + bias_ref[...]
    ii = lax.broadcasted_iota(jnp.int32, (BT, E), 1)
    m1 = jnp.max(sc, axis=1, keepdims=True)
    i1 = jnp.min(jnp.where(sc >= m1, ii, E), axis=1, keepdims=True)
    sc2 = jnp.where(ii == i1, -jnp.inf, sc)
    m2 = jnp.max(sc2, axis=1, keepdims=True)
    i2 = jnp.min(jnp.where(sc2 >= m2, ii, E), axis=1, keepdims=True)
    s1 = jnp.sum(jnp.where(ii == i1, s, 0.0), axis=1, keepdims=True)
    s2 = jnp.sum(jnp.where(ii == i2, s, 0.0), axis=1, keepdims=True)
    denom = s1 + s2
    w1 = s1 / denom
    w2 = s2 / denom

    xb = x.astype(jnp.bfloat16)

    sgu = jnp.dot(xb, sgu_ref[...], preferred_element_type=jnp.float32)
    sh = (jax.nn.silu(sgu[:, :I]) * sgu[:, I:]).astype(jnp.bfloat16)
    acc = jnp.dot(sh, sd_ref[...], preferred_element_type=jnp.float32)

    for e in range(E):
        w_e = jnp.where(i1 == e, w1, 0.0) + jnp.where(i2 == e, w2, 0.0)
        gu = jnp.dot(xb, wgu_ref[e], preferred_element_type=jnp.float32)
        h = (jax.nn.silu(gu[:, :I]) * gu[:, I:]).astype(jnp.bfloat16)
        oe = jnp.dot(h, wd_ref[e], preferred_element_type=jnp.float32)
        acc = acc + w_e * oe

    out_ref[...] = acc


@functools.partial(jax.jit, static_argnames=("interpret",))
def _moe_dense(x, gate_weight, bias, w_gate_up, w_down, sgu, sd, interpret=False):
    grid = (T // BT,)
    return pl.pallas_call(
        _dense_body,
        grid=grid,
        in_specs=[
            pl.BlockSpec((BT, H), lambda t: (t, 0)),
            pl.BlockSpec((E, H), lambda t: (0, 0)),
            pl.BlockSpec((1, E), lambda t: (0, 0)),
            pl.BlockSpec((E, H, 2 * I), lambda t: (0, 0, 0)),
            pl.BlockSpec((E, I, H), lambda t: (0, 0, 0)),
            pl.BlockSpec((H, 2 * I), lambda t: (0, 0)),
            pl.BlockSpec((I, H), lambda t: (0, 0)),
        ],
        out_specs=pl.BlockSpec((BT, H), lambda t: (t, 0)),
        out_shape=jax.ShapeDtypeStruct((T, H), jnp.float32),
        compiler_params=pltpu.CompilerParams(
            dimension_semantics=("arbitrary",),
        ),
        interpret=interpret,
    )(x, gate_weight, bias, w_gate_up, w_down, sgu, sd)


def kernel(hidden_states, gate_weight, correction_bias, w_gate_up, w_down,
           shared_gate_up, shared_down):
    bf = jnp.bfloat16
    return _moe_dense(hidden_states, gate_weight, correction_bias,
                      w_gate_up.astype(bf), w_down.astype(bf),
                      shared_gate_up.astype(bf), shared_down.astype(bf))
